# Initial kernel scaffold; baseline (speedup 1.0000x reference)
import jax
import jax.numpy as jnp
from jax.experimental import pallas as pl


def kernel(hidden, edge_index, We, be, W1, b1, W2, b2, W3, b3):
    raise NotImplementedError("write your pallas kernel here")


# speedup vs baseline: 4.0667x; 4.0667x over previous
import functools

import jax
import jax.numpy as jnp
from jax import lax
from jax.experimental import pallas as pl
from jax.experimental.pallas import tpu as pltpu
from jax.experimental.pallas import tpu_sc as plsc

N = 10000
E = 320000
H = 128

NP = 10240
NC = 1
NS = 16
NW = NC * NS
SUB = 64
KI = 40
NBLK = 8
NSUB = KI * NBLK
EP = NW * NSUB * SUB
ROWS_PER_TILE = NP // NS


def _pre_body(h_ref, wa_ref, wb_ref, be_ref, a_ref, b_ref):
    h = h_ref[...]
    a_ref[...] = jnp.dot(h, wa_ref[...], preferred_element_type=jnp.float32)
    b_ref[...] = (jnp.dot(h, wb_ref[...], preferred_element_type=jnp.float32)
                  + be_ref[...])


def _pre_ab(hidden_pad, wa, wb, be2):
    blk = NP // 5
    return pl.pallas_call(
        _pre_body,
        grid=(5,),
        in_specs=[
            pl.BlockSpec((blk, H), lambda i: (i, 0)),
            pl.BlockSpec((H, H), lambda i: (0, 0)),
            pl.BlockSpec((H, H), lambda i: (0, 0)),
            pl.BlockSpec((1, H), lambda i: (0, 0)),
        ],
        out_specs=[
            pl.BlockSpec((blk, H), lambda i: (i, 0)),
            pl.BlockSpec((blk, H), lambda i: (i, 0)),
        ],
        out_shape=[
            jax.ShapeDtypeStruct((NP, H), jnp.float32),
            jax.ShapeDtypeStruct((NP, H), jnp.float32),
        ],
    )(hidden_pad, wa, wb, be2)


def _zero_buf(buf):
    zeros = jnp.zeros((16,), jnp.float32)

    def row(i, _):
        for c in range(H // 16):
            buf[i, pl.ds(c * 16, 16)] = zeros
        return 0

    lax.fori_loop(0, SUB, row, 0)


def _relu_add(buf_a, buf_b):
    def row(i, _):
        for c in range(H // 16):
            sl = pl.ds(c * 16, 16)
            buf_a[i, sl] = jnp.maximum(buf_a[i, sl] + buf_b[i, sl], 0.0)
        return 0

    lax.fori_loop(0, SUB, row, 0)


def _sc_body(a_hbm, b_hbm, src_hbm, dst_hbm, out_hbm,
             src_v, dst_v, ab0, ab1, bb0, bb1, agg_sh,
             sa0, sa1, sb0, sb1):
    sid = lax.axis_index("s")
    wid = sid

    abuf = (ab0, ab1)
    bbuf = (bb0, bb1)
    sema = (sa0, sa1)
    semb = (sb0, sb1)

    _zero_buf(ab0)
    for kk in range(ROWS_PER_TILE // SUB):
        pltpu.sync_copy(
            ab0, agg_sh.at[pl.ds(sid * ROWS_PER_TILE + kk * SUB, SUB)])
    plsc.subcore_barrier()

    def start(j, k):
        pltpu.async_copy(a_hbm.at[src_v.at[j]], abuf[k], sema[k])
        pltpu.async_copy(b_hbm.at[dst_v.at[j]], bbuf[k], semb[k])

    def wait(j, k):
        pltpu.make_async_copy(a_hbm.at[src_v.at[j]], abuf[k], sema[k]).wait()
        pltpu.make_async_copy(b_hbm.at[dst_v.at[j]], bbuf[k], semb[k]).wait()

    def step(j, k, j_next):
        if j_next is not None:
            start(j_next, 1 - k)
        wait(j, k)
        _relu_add(abuf[k], bbuf[k])
        pltpu.sync_copy(abuf[k], agg_sh.at[dst_v.at[j]], add=True)

    def block(kb, _):
        pltpu.sync_copy(src_hbm.at[wid, pl.ds(kb * KI, KI)], src_v)
        pltpu.sync_copy(dst_hbm.at[wid, pl.ds(kb * KI, KI)], dst_v)

        start(0, 0)

        def pair(j2, _):
            j = j2 * 2
            step(j, 0, j + 1)
            step(j + 1, 1, j + 2)
            return 0

        lax.fori_loop(0, KI // 2 - 1, pair, 0)
        step(KI - 2, 0, KI - 1)
        step(KI - 1, 1, None)
        return 0

    lax.fori_loop(0, NBLK, block, 0)

    plsc.subcore_barrier()
    base = sid * ROWS_PER_TILE
    pltpu.sync_copy(agg_sh.at[pl.ds(base, ROWS_PER_TILE)],
                    out_hbm.at[pl.ds(base, ROWS_PER_TILE)])


def _sc_aggregate(a_tab, b_tab, src_r, dst_r):
    mesh = plsc.VectorSubcoreMesh(core_axis_name="c", subcore_axis_name="s",
                                  num_cores=NC)
    fn = functools.partial(
        pl.kernel,
        out_type=jax.ShapeDtypeStruct((NP, H), jnp.float32),
        mesh=mesh,
        scratch_types=[
            pltpu.VMEM((KI, SUB), jnp.int32),
            pltpu.VMEM((KI, SUB), jnp.int32),
            pltpu.VMEM((SUB, H), jnp.float32),
            pltpu.VMEM((SUB, H), jnp.float32),
            pltpu.VMEM((SUB, H), jnp.float32),
            pltpu.VMEM((SUB, H), jnp.float32),
            pltpu.VMEM_SHARED((NP, H), jnp.float32),
            pltpu.SemaphoreType.DMA,
            pltpu.SemaphoreType.DMA,
            pltpu.SemaphoreType.DMA,
            pltpu.SemaphoreType.DMA,
        ],
    )(_sc_body)
    return fn(a_tab, b_tab, src_r, dst_r)


def _post_body(h_ref, agg_ref, w1a_ref, w1b_ref, b1_ref,
               w2_ref, b2_ref, w3_ref, b3_ref, o_ref):
    agg = agg_ref[...]
    h1 = jnp.tanh(
        jnp.dot(h_ref[...], w1a_ref[...], preferred_element_type=jnp.float32)
        + jnp.dot(agg, w1b_ref[...], preferred_element_type=jnp.float32)
        + b1_ref[...])
    h2 = jnp.tanh(
        jnp.dot(h1, w2_ref[...], preferred_element_type=jnp.float32)
        + b2_ref[...])
    o_ref[...] = (jnp.dot(h2, w3_ref[...], preferred_element_type=jnp.float32)
                  + b3_ref[...])


def _post_mlp(hidden, agg, w1a, w1b, b1, w2, b2, w3, b3):
    blk = 2000
    row_spec = pl.BlockSpec((blk, H), lambda i: (i, 0))
    w_spec = pl.BlockSpec((H, H), lambda i: (0, 0))
    b_spec = pl.BlockSpec((1, H), lambda i: (0, 0))
    return pl.pallas_call(
        _post_body,
        grid=(N // blk,),
        in_specs=[row_spec, row_spec,
                  w_spec, w_spec, b_spec, w_spec, b_spec, w_spec, b_spec],
        out_specs=row_spec,
        out_shape=jax.ShapeDtypeStruct((N, H), jnp.float32),
    )(hidden, agg, w1a, w1b, b1, w2, b2, w3, b3)


@jax.jit
def kernel(hidden, edge_index, We, be, W1, b1, W2, b2, W3, b3):
    ei = edge_index.astype(jnp.int32)
    pad_e = EP - E
    src_r = jnp.concatenate(
        [ei[0], jnp.full((pad_e,), N, jnp.int32)]).reshape(NW, NSUB, SUB)
    dst_r = jnp.concatenate(
        [ei[1], jnp.full((pad_e,), N, jnp.int32)]).reshape(NW, NSUB, SUB)
    hidden_pad = jnp.concatenate(
        [hidden, jnp.zeros((NP - N, H), jnp.float32)], axis=0)

    a_tab, b_tab = _pre_ab(hidden_pad, We[:H], We[H:], be.reshape(1, H))
    agg = _sc_aggregate(a_tab, b_tab, src_r, dst_r)
    return _post_mlp(hidden, agg,
                     W1[:H], W1[H:], b1.reshape(1, H),
                     W2, b2.reshape(1, H), W3, b3.reshape(1, H))
